# baseline (speedup 1.0000x reference)
import functools

import jax
import jax.numpy as jnp
from jax import lax
from jax.experimental import pallas as pl
from jax.experimental.pallas import tpu as pltpu
from jax.experimental.pallas import tpu_sc as plsc

NUM_REL = 32
D = 16
N_IDS = 3200000
NC = 2
NS = 16
NW = NC * NS
NBLK = N_IDS // 128
CB = 25
T_CHUNKS = NBLK // CB

_mesh = plsc.VectorSubcoreMesh(core_axis_name="c", subcore_axis_name="s")


@functools.partial(
    pl.kernel,
    mesh=_mesh,
    out_type=jax.ShapeDtypeStruct((2, NBLK, 8, 128), jnp.float32),
    compiler_params=pltpu.CompilerParams(
        use_tc_tiling_on_sc=False, needs_layout_passes=False,
        skip_device_barrier=True),
    scratch_types=[
        pltpu.VMEM((NUM_REL * D,), jnp.float32),
        pltpu.VMEM((NUM_REL * D * 16,), jnp.float32),
        pltpu.VMEM((CB * 128,), jnp.int32),
        pltpu.VMEM((CB * 128,), jnp.int32),
        pltpu.VMEM((2, CB, 8, 128), jnp.float32),
        pltpu.VMEM((2, CB, 8, 128), jnp.float32),
        pltpu.SemaphoreType.DMA,
        pltpu.SemaphoreType.DMA,
        pltpu.SemaphoreType.DMA,
        pltpu.SemaphoreType.DMA,
    ],
)
def _lookup(ids_hbm, table_hbm, out_hbm, table_v, m_v,
            idx0, idx1, ob0, ob1, isem0, isem1, osem0, osem1):
    wid = lax.axis_index("s") * NC + lax.axis_index("c")
    idx_b = (idx0, idx1)
    ob_b = (ob0, ob1)
    isem = (isem0, isem1)
    osem = (osem0, osem1)
    count = (T_CHUNKS - wid + NW - 1) // NW

    def ids_slice(g):
        return ids_hbm.at[pl.ds((wid + g * NW) * CB * 128, CB * 128)]

    pltpu.async_copy(ids_slice(0), idx0, isem0)
    pltpu.sync_copy(table_hbm, table_v)
    iota = lax.iota(jnp.int32, 16)

    def build(e, carry):
        vals = plsc.load_gather(table_v, [jnp.full((16,), e, jnp.int32)])
        m_v[pl.ds(e * 16, 16)] = vals
        return carry

    lax.fori_loop(0, NUM_REL * D, build, 0)

    def pair(g2, carry):
        for p in range(2):
            g = g2 * 2 + p

            @pl.when(g < count)
            def _chunk():
                cb0 = (wid + g * NW) * CB
                idx_v, obuf = idx_b[p], ob_b[p]

                @pl.when(g + 1 < count)
                def _prefetch():
                    pltpu.async_copy(ids_slice(g + 1), idx_b[1 - p],
                                     isem[1 - p])

                pltpu.make_async_copy(ids_slice(g), idx_v, isem[p]).wait()

                @pl.when(g >= 2)
                def _drain():
                    pltpu.make_async_copy(
                        obuf.at[0], out_hbm.at[0, pl.ds(0, CB)],
                        osem[p]).wait()
                    pltpu.make_async_copy(
                        obuf.at[1], out_hbm.at[1, pl.ds(0, CB)],
                        osem[p]).wait()

                @plsc.parallel_loop(0, CB, unroll=2)
                def block(b):
                    ivs = [idx_v[pl.ds(b * 128 + s * 16, 16)] * 256
                           for s in range(8)]
                    for j in range(D):
                        jl = iota + j * 16
                        for s in range(8):
                            vals = plsc.load_gather(m_v, [ivs[s] + jl])
                            obuf[j // 8, b, j % 8, pl.ds(s * 16, 16)] = vals

                pltpu.async_copy(obuf.at[0], out_hbm.at[0, pl.ds(cb0, CB)],
                                 osem[p])
                pltpu.async_copy(obuf.at[1], out_hbm.at[1, pl.ds(cb0, CB)],
                                 osem[p])
        return carry

    lax.fori_loop(0, (count + 1) // 2, pair, 0)
    for p in range(2):
        pltpu.make_async_copy(ob_b[p].at[0], out_hbm.at[0, pl.ds(0, CB)],
                              osem[p]).wait()
        pltpu.make_async_copy(ob_b[p].at[1], out_hbm.at[1, pl.ds(0, CB)],
                              osem[p]).wait()


def kernel(relation_ids, embeddings):
    f4 = _lookup(relation_ids.astype(jnp.int32),
                 embeddings.reshape(NUM_REL * D))
    return f4.transpose(1, 3, 0, 2).reshape(N_IDS, D)


# speedup vs baseline: 1.1928x; 1.1928x over previous
import functools

import jax
import jax.numpy as jnp
from jax import lax
from jax.experimental import pallas as pl
from jax.experimental.pallas import tpu as pltpu
from jax.experimental.pallas import tpu_sc as plsc

NUM_REL = 32
D = 16
N_IDS = 3200000
NC = 2
NS = 16
NW = NC * NS
NBLK = N_IDS // 128
CB = 25
T_CHUNKS = NBLK // CB

_mesh = plsc.VectorSubcoreMesh(core_axis_name="c", subcore_axis_name="s")


@functools.partial(
    pl.kernel,
    mesh=_mesh,
    out_type=jax.ShapeDtypeStruct((2, NBLK, 8, 128), jnp.float32),
    compiler_params=pltpu.CompilerParams(
        use_tc_tiling_on_sc=False, needs_layout_passes=False,
        skip_device_barrier=True),
    scratch_types=[
        pltpu.VMEM((NUM_REL * D,), jnp.float32),
        pltpu.VMEM((NUM_REL * D * 16,), jnp.float32),
        pltpu.VMEM((CB * 128,), jnp.int32),
        pltpu.VMEM((CB * 128,), jnp.int32),
        pltpu.VMEM((2, CB, 8, 128), jnp.float32),
        pltpu.VMEM((2, CB, 8, 128), jnp.float32),
        pltpu.SemaphoreType.DMA,
        pltpu.SemaphoreType.DMA,
        pltpu.SemaphoreType.DMA,
        pltpu.SemaphoreType.DMA,
    ],
)
def _lookup(ids_hbm, table_hbm, out_hbm, table_v, m_v,
            idx0, idx1, ob0, ob1, isem0, isem1, osem0, osem1):
    wid = lax.axis_index("s") * NC + lax.axis_index("c")
    idx_b = (idx0, idx1)
    ob_b = (ob0, ob1)
    isem = (isem0, isem1)
    osem = (osem0, osem1)
    count = (T_CHUNKS - wid + NW - 1) // NW

    def ids_slice(g):
        return ids_hbm.at[pl.ds((wid + g * NW) * CB * 128, CB * 128)]

    pltpu.async_copy(ids_slice(0), idx0, isem0)
    pltpu.sync_copy(table_hbm, table_v)
    iota = lax.iota(jnp.int32, 16)

    def build(e, carry):
        vals = plsc.load_gather(table_v, [jnp.full((16,), e, jnp.int32)])
        m_v[pl.ds(e * 16, 16)] = vals
        return carry

    lax.fori_loop(0, NUM_REL * D, build, 0)

    def pair(g2, carry):
        for p in range(2):
            g = g2 * 2 + p

            @pl.when(g < count)
            def _chunk():
                cb0 = (wid + g * NW) * CB
                idx_v, obuf = idx_b[p], ob_b[p]

                @pl.when(g + 1 < count)
                def _prefetch():
                    pltpu.async_copy(ids_slice(g + 1), idx_b[1 - p],
                                     isem[1 - p])

                pltpu.make_async_copy(ids_slice(g), idx_v, isem[p]).wait()

                @pl.when(g >= 2)
                def _drain():
                    pltpu.make_async_copy(
                        obuf.at[0], out_hbm.at[0, pl.ds(0, CB)],
                        osem[p]).wait()
                    pltpu.make_async_copy(
                        obuf.at[1], out_hbm.at[1, pl.ds(0, CB)],
                        osem[p]).wait()

                @plsc.parallel_loop(0, CB, unroll=1)
                def block(b):
                    ivs = [idx_v[pl.ds(b * 128 + s * 16, 16)] * 256
                           for s in range(8)]
                    for j in range(D):
                        jl = iota + j * 16
                        for s in range(8):
                            vals = plsc.load_gather(m_v, [ivs[s] + jl])
                            obuf[j // 8, b, j % 8, pl.ds(s * 16, 16)] = vals

                pltpu.async_copy(obuf.at[0], out_hbm.at[0, pl.ds(cb0, CB)],
                                 osem[p])
                pltpu.async_copy(obuf.at[1], out_hbm.at[1, pl.ds(cb0, CB)],
                                 osem[p])
        return carry

    lax.fori_loop(0, (count + 1) // 2, pair, 0)
    for p in range(2):
        pltpu.make_async_copy(ob_b[p].at[0], out_hbm.at[0, pl.ds(0, CB)],
                              osem[p]).wait()
        pltpu.make_async_copy(ob_b[p].at[1], out_hbm.at[1, pl.ds(0, CB)],
                              osem[p]).wait()


def kernel(relation_ids, embeddings):
    f4 = _lookup(relation_ids.astype(jnp.int32),
                 embeddings.reshape(NUM_REL * D))
    return f4.transpose(1, 3, 0, 2).reshape(N_IDS, D)


# speedup vs baseline: 1.2453x; 1.0440x over previous
import functools

import jax
import jax.numpy as jnp
from jax import lax
from jax.experimental import pallas as pl
from jax.experimental.pallas import tpu as pltpu
from jax.experimental.pallas import tpu_sc as plsc

NUM_REL = 32
D = 16
N_IDS = 3200000
NC = 2
NS = 16
NW = NC * NS
NBLK = N_IDS // 128
CB = 25
T_CHUNKS = NBLK // CB

_mesh = plsc.VectorSubcoreMesh(core_axis_name="c", subcore_axis_name="s")


@functools.partial(
    pl.kernel,
    mesh=_mesh,
    out_type=jax.ShapeDtypeStruct((2, NBLK, 8, 128), jnp.float32),
    compiler_params=pltpu.CompilerParams(
        use_tc_tiling_on_sc=False, needs_layout_passes=False,
        skip_device_barrier=True),
    scratch_types=[
        pltpu.VMEM((NUM_REL * D,), jnp.float32),
        pltpu.VMEM((NUM_REL * D * 16,), jnp.float32),
        pltpu.VMEM((CB * 128,), jnp.int32),
        pltpu.VMEM((CB * 128,), jnp.int32),
        pltpu.VMEM((2, CB, 8, 128), jnp.float32),
        pltpu.VMEM((2, CB, 8, 128), jnp.float32),
        pltpu.SemaphoreType.DMA,
        pltpu.SemaphoreType.DMA,
        pltpu.SemaphoreType.DMA,
        pltpu.SemaphoreType.DMA,
        pltpu.SemaphoreType.DMA,
        pltpu.SemaphoreType.DMA,
    ],
)
def _lookup(ids_hbm, table_hbm, out_hbm, table_v, m_v,
            idx0, idx1, ob0, ob1, isem0, isem1,
            osem00, osem01, osem10, osem11):
    wid = lax.axis_index("s") * NC + lax.axis_index("c")
    idx_b = (idx0, idx1)
    ob_b = (ob0, ob1)
    isem = (isem0, isem1)
    osem = ((osem00, osem01), (osem10, osem11))
    count = (T_CHUNKS - wid + NW - 1) // NW

    def ids_slice(g):
        return ids_hbm.at[pl.ds((wid + g * NW) * CB * 128, CB * 128)]

    pltpu.async_copy(ids_slice(0), idx0, isem0)
    pltpu.sync_copy(table_hbm, table_v)
    iota = lax.iota(jnp.int32, 16)

    def build(e, carry):
        vals = plsc.load_gather(table_v, [jnp.full((16,), e, jnp.int32)])
        m_v[pl.ds(e * 16, 16)] = vals
        return carry

    lax.fori_loop(0, NUM_REL * D, build, 0)

    def pair(g2, carry):
        for p in range(2):
            g = g2 * 2 + p

            @pl.when(g < count)
            def _chunk():
                cb0 = (wid + g * NW) * CB
                idx_v, obuf = idx_b[p], ob_b[p]

                @pl.when(g + 1 < count)
                def _prefetch():
                    pltpu.async_copy(ids_slice(g + 1), idx_b[1 - p],
                                     isem[1 - p])

                pltpu.make_async_copy(ids_slice(g), idx_v, isem[p]).wait()

                for r in range(2):
                    @pl.when(g >= 2)
                    def _drain():
                        pltpu.make_async_copy(
                            obuf.at[r], out_hbm.at[r, pl.ds(0, CB)],
                            osem[p][r]).wait()

                    @plsc.parallel_loop(0, CB, unroll=1)
                    def block(b):
                        ivs = [idx_v[pl.ds(b * 128 + s * 16, 16)] * 256
                               for s in range(8)]
                        for jr in range(8):
                            jl = iota + (r * 8 + jr) * 16
                            for s in range(8):
                                vals = plsc.load_gather(m_v, [ivs[s] + jl])
                                obuf[r, b, jr, pl.ds(s * 16, 16)] = vals

                    pltpu.async_copy(obuf.at[r],
                                     out_hbm.at[r, pl.ds(cb0, CB)],
                                     osem[p][r])
        return carry

    lax.fori_loop(0, (count + 1) // 2, pair, 0)
    for p in range(2):
        for r in range(2):
            pltpu.make_async_copy(ob_b[p].at[r], out_hbm.at[r, pl.ds(0, CB)],
                                  osem[p][r]).wait()


def kernel(relation_ids, embeddings):
    f4 = _lookup(relation_ids.astype(jnp.int32),
                 embeddings.reshape(NUM_REL * D))
    return f4.transpose(1, 3, 0, 2).reshape(N_IDS, D)
